# baseline (speedup 1.0000x reference)
import jax
import jax.numpy as jnp
from jax import lax
from jax.experimental import pallas as pl


def _copy_body(k_ref, v_ref, ko_ref, vo_ref):
    ko_ref[...] = k_ref[...]
    vo_ref[...] = v_ref[...]


def kernel(key_states, value_states, k_cache, v_cache, layer_idx):
    B, H, S, D = key_states.shape
    D2 = D // 2
    k = lax.bitcast_convert_type(key_states.reshape(H, S, D2, 2), jnp.int32)
    v = lax.bitcast_convert_type(value_states.reshape(H, S, D2, 2), jnp.int32)
    HB = 4
    spec = pl.BlockSpec((HB, S, D2), lambda h: (h, 0, 0))
    ko, vo = pl.pallas_call(
        _copy_body,
        grid=(H // HB,),
        in_specs=[spec, spec],
        out_specs=[spec, spec],
        out_shape=[jax.ShapeDtypeStruct((H, S, D2), jnp.int32)] * 2,
    )(k, v)
    ko = lax.bitcast_convert_type(ko, jnp.float16).reshape(B, H, S, D)
    vo = lax.bitcast_convert_type(vo, jnp.float16).reshape(B, H, S, D)
    return ko, vo
